# baseline (speedup 1.0000x reference)
import functools

import jax
import jax.numpy as jnp
from jax.experimental import pallas as pl
from jax.experimental.pallas import tpu as pltpu

_DOT_DIMS = (((1,), (0,)), ((), ()))


def _phased_kernel(xa_ref, x_ref, w_ref, b_ref, o_ref, wb_ref, acc_ref,
                   *, nc, kc, pa, nt):
    i = pl.program_id(1)

    @pl.when(i < nc)
    def _phase_a():
        wc = w_ref[...].astype(jnp.bfloat16)
        wb_ref[pl.ds(i * kc, kc), :] = wc
        part = jax.lax.dot_general(xa_ref[...].astype(jnp.bfloat16), wc,
                                   dimension_numbers=_DOT_DIMS,
                                   preferred_element_type=jnp.float32)

        @pl.when(i == 0)
        def _():
            acc_ref[...] = part

        @pl.when(i > 0)
        def _():
            acc_ref[...] += part

    @pl.when(jnp.logical_and(i >= nc, i < nc + nt - pa))
    def _phase_b():
        acc = jax.lax.dot_general(x_ref[...].astype(jnp.bfloat16),
                                  wb_ref[...],
                                  dimension_numbers=_DOT_DIMS,
                                  preferred_element_type=jnp.float32)
        o_ref[...] = acc + b_ref[...]

    @pl.when(i >= nc + nt - pa)
    def _tail():
        r = (i - (nc + nt - pa)) * o_ref.shape[0]
        o_ref[...] = acc_ref[pl.ds(r, o_ref.shape[0]), :] + b_ref[...]


def _forward(x, w, b, *, tm, kc, pa):
    B, K = x.shape
    _, N = w.shape
    tn = N // 2
    nc = K // kc
    nt = B // tm
    grid_i = nc + nt

    kern = functools.partial(_phased_kernel, nc=nc, kc=kc, pa=pa, nt=nt)

    def xa_map(j, i):
        return (0, jnp.minimum(i, nc - 1))

    def x_map(j, i):
        return (jnp.clip(i - (nc - pa), pa, nt - 1), 0)

    def w_map(j, i):
        return (jnp.minimum(i, nc - 1), j)

    def o_map(j, i):
        return (jnp.where(i >= nc + nt - pa,
                          i - (nc + nt - pa),
                          jnp.clip(i - (nc - pa), pa, nt - 1)), j)

    return pl.pallas_call(
        kern,
        out_shape=jax.ShapeDtypeStruct((B, N), x.dtype),
        grid=(2, grid_i),
        in_specs=[
            pl.BlockSpec((pa * tm, kc), xa_map),
            pl.BlockSpec((tm, K), x_map),
            pl.BlockSpec((kc, tn), w_map),
            pl.BlockSpec((1, tn), lambda j, i: (0, j)),
        ],
        out_specs=pl.BlockSpec((tm, tn), o_map),
        scratch_shapes=[
            pltpu.VMEM((K, tn), jnp.bfloat16),
            pltpu.VMEM((pa * tm, tn), jnp.float32),
        ],
        compiler_params=pltpu.CompilerParams(
            dimension_semantics=("parallel", "arbitrary"),
            vmem_limit_bytes=60 << 20,
        ),
    )(x, x, w, b)


def kernel(x, w, b):
    B, K = x.shape
    K2, N = w.shape
    assert K == K2, (K, K2)
    assert B % 256 == 0 and K % 512 == 0 and N % 512 == 0, (B, K, N)
    return _forward(x, w, b, tm=256, kc=512, pa=4)


# speedup vs baseline: 1.0118x; 1.0118x over previous
import functools

import jax
import jax.numpy as jnp
from jax.experimental import pallas as pl
from jax.experimental.pallas import tpu as pltpu

_DOT_DIMS = (((1,), (0,)), ((), ()))


def _phased_kernel(xa_ref, x_ref, w_ref, b_ref, o_ref, wb_ref, acc_ref,
                   *, nc, kc, pa, nt):
    i = pl.program_id(1)

    @pl.when(i < nc)
    def _phase_a():
        wc = w_ref[...].astype(jnp.bfloat16)
        wb_ref[pl.ds(i * kc, kc), :] = wc
        part = jax.lax.dot_general(xa_ref[...].astype(jnp.bfloat16), wc,
                                   dimension_numbers=_DOT_DIMS,
                                   preferred_element_type=jnp.float32)

        @pl.when(i == 0)
        def _():
            acc_ref[...] = part

        @pl.when(i > 0)
        def _():
            acc_ref[...] += part

    @pl.when(jnp.logical_and(i >= nc, i < nc + nt - pa))
    def _phase_b():
        acc = jax.lax.dot_general(x_ref[...].astype(jnp.bfloat16),
                                  wb_ref[...],
                                  dimension_numbers=_DOT_DIMS,
                                  preferred_element_type=jnp.float32)
        o_ref[...] = acc + b_ref[...]

    @pl.when(i >= nc + nt - pa)
    def _tail():
        r = (i - (nc + nt - pa)) * o_ref.shape[0]
        o_ref[...] = acc_ref[pl.ds(r, o_ref.shape[0]), :] + b_ref[...]


def _forward(x, w, b, *, tm, kc, pa):
    B, K = x.shape
    _, N = w.shape
    tn = N // 2
    nc = K // kc
    nt = B // tm
    grid_i = nc + nt

    kern = functools.partial(_phased_kernel, nc=nc, kc=kc, pa=pa, nt=nt)

    def xa_map(j, i):
        return (0, jnp.minimum(i, nc - 1))

    def x_map(j, i):
        return (jnp.clip(i - (nc - pa), pa, nt - 1), 0)

    def w_map(j, i):
        return (jnp.minimum(i, nc - 1), j)

    def o_map(j, i):
        return (jnp.where(i >= nc + nt - pa,
                          i - (nc + nt - pa),
                          jnp.clip(i - (nc - pa), pa, nt - 1)), j)

    return pl.pallas_call(
        kern,
        out_shape=jax.ShapeDtypeStruct((B, N), x.dtype),
        grid=(2, grid_i),
        in_specs=[
            pl.BlockSpec((pa * tm, kc), xa_map),
            pl.BlockSpec((tm, K), x_map),
            pl.BlockSpec((kc, tn), w_map),
            pl.BlockSpec((1, tn), lambda j, i: (0, j)),
        ],
        out_specs=pl.BlockSpec((tm, tn), o_map),
        scratch_shapes=[
            pltpu.VMEM((K, tn), jnp.bfloat16),
            pltpu.VMEM((pa * tm, tn), jnp.float32),
        ],
        compiler_params=pltpu.CompilerParams(
            dimension_semantics=("parallel", "arbitrary"),
            vmem_limit_bytes=60 << 20,
        ),
    )(x, x, w, b)


def kernel(x, w, b):
    B, K = x.shape
    K2, N = w.shape
    assert K == K2, (K, K2)
    assert B % 256 == 0 and K % 512 == 0 and N % 512 == 0, (B, K, N)
    return _forward(x, w, b, tm=256, kc=512, pa=2)


# speedup vs baseline: 1.0254x; 1.0134x over previous
import functools

import jax
import jax.numpy as jnp
from jax.experimental import pallas as pl
from jax.experimental.pallas import tpu as pltpu

_DOT_DIMS = (((1,), (0,)), ((), ()))


def _phased_kernel(xa_ref, x_ref, w_ref, b_ref, o_ref, wb_ref, acc_ref,
                   *, nc, kc, pa, nt):
    i = pl.program_id(1)

    @pl.when(i < nc)
    def _phase_a():
        wc = w_ref[...].astype(jnp.bfloat16)
        wb_ref[pl.ds(i * kc, kc), :] = wc
        part = jax.lax.dot_general(xa_ref[...].astype(jnp.bfloat16), wc,
                                   dimension_numbers=_DOT_DIMS,
                                   preferred_element_type=jnp.float32)

        @pl.when(i == 0)
        def _():
            acc_ref[...] = part

        @pl.when(i > 0)
        def _():
            acc_ref[...] += part

    @pl.when(jnp.logical_and(i >= nc, i < nc + nt - pa))
    def _phase_b():
        acc = jax.lax.dot_general(x_ref[...].astype(jnp.bfloat16),
                                  wb_ref[...],
                                  dimension_numbers=_DOT_DIMS,
                                  preferred_element_type=jnp.float32)
        o_ref[...] = acc + b_ref[...]

    @pl.when(i >= nc + nt - pa)
    def _tail():
        r = (i - (nc + nt - pa)) * o_ref.shape[0]
        o_ref[...] = acc_ref[pl.ds(r, o_ref.shape[0]), :] + b_ref[...]


def _forward(x, w, b, *, tm, kc, pa):
    B, K = x.shape
    _, N = w.shape
    tn = N // 2
    nc = K // kc
    nt = B // tm
    grid_i = nc + nt

    kern = functools.partial(_phased_kernel, nc=nc, kc=kc, pa=pa, nt=nt)

    def xa_map(j, i):
        return (0, jnp.minimum(i, nc - 1))

    def x_map(j, i):
        return (jnp.clip(i - (nc - pa), pa, nt - 1), 0)

    def w_map(j, i):
        return (jnp.minimum(i, nc - 1), j)

    def o_map(j, i):
        return (jnp.where(i >= nc + nt - pa,
                          i - (nc + nt - pa),
                          jnp.clip(i - (nc - pa), pa, nt - 1)), j)

    return pl.pallas_call(
        kern,
        out_shape=jax.ShapeDtypeStruct((B, N), x.dtype),
        grid=(2, grid_i),
        in_specs=[
            pl.BlockSpec((pa * tm, kc), xa_map),
            pl.BlockSpec((tm, K), x_map),
            pl.BlockSpec((kc, tn), w_map),
            pl.BlockSpec((1, tn), lambda j, i: (0, j)),
        ],
        out_specs=pl.BlockSpec((tm, tn), o_map),
        scratch_shapes=[
            pltpu.VMEM((K, tn), jnp.bfloat16),
            pltpu.VMEM((pa * tm, tn), jnp.float32),
        ],
        compiler_params=pltpu.CompilerParams(
            dimension_semantics=("parallel", "arbitrary"),
            vmem_limit_bytes=64 << 20,
        ),
    )(x, x, w, b)


def kernel(x, w, b):
    B, K = x.shape
    K2, N = w.shape
    assert K == K2, (K, K2)
    assert B % 256 == 0 and K % 512 == 0 and N % 512 == 0, (B, K, N)
    return _forward(x, w, b, tm=256, kc=1024, pa=3)


# speedup vs baseline: 1.0720x; 1.0454x over previous
import functools

import jax
import jax.numpy as jnp
from jax.experimental import pallas as pl
from jax.experimental.pallas import tpu as pltpu

_DOT_DIMS = (((1,), (0,)), ((), ()))


def _phased_kernel(xa_ref, x_ref, w_ref, b_ref, o_ref, wb_ref, acc_ref,
                   *, nc, kc, pa, nt):
    i = pl.program_id(1)

    @pl.when(i < nc)
    def _phase_a():
        wc = w_ref[...].astype(jnp.bfloat16)
        wb_ref[pl.ds(i * kc, kc), :] = wc
        part = jax.lax.dot_general(xa_ref[...].astype(jnp.bfloat16), wc,
                                   dimension_numbers=_DOT_DIMS,
                                   preferred_element_type=jnp.float32)

        @pl.when(i == 0)
        def _():
            acc_ref[...] = part

        @pl.when(i > 0)
        def _():
            acc_ref[...] += part

    @pl.when(jnp.logical_and(i >= nc, i < nc + nt - pa))
    def _phase_b():
        acc = jax.lax.dot_general(x_ref[...].astype(jnp.bfloat16),
                                  wb_ref[...],
                                  dimension_numbers=_DOT_DIMS,
                                  preferred_element_type=jnp.float32)
        o_ref[...] = acc + b_ref[...]

    @pl.when(i >= nc + nt - pa)
    def _tail():
        r = (i - (nc + nt - pa)) * o_ref.shape[0]
        o_ref[...] = acc_ref[pl.ds(r, o_ref.shape[0]), :] + b_ref[...]


def _phased_kernel_noacc(xa_ref, x_ref, w_ref, b_ref, o_ref, wb_ref,
                         *, nc, kc, nt):
    i = pl.program_id(1)

    @pl.when(i < nc)
    def _phase_a():
        wc = w_ref[...].astype(jnp.bfloat16)
        wb_ref[pl.ds(i * kc, kc), :] = wc
        part = jax.lax.dot_general(xa_ref[...].astype(jnp.bfloat16), wc,
                                   dimension_numbers=_DOT_DIMS,
                                   preferred_element_type=jnp.float32)

        @pl.when(i == 0)
        def _():
            o_ref[...] = part + b_ref[...]

        @pl.when(i > 0)
        def _():
            o_ref[...] += part

    @pl.when(i >= nc)
    def _phase_b():
        acc = jax.lax.dot_general(x_ref[...].astype(jnp.bfloat16),
                                  wb_ref[...],
                                  dimension_numbers=_DOT_DIMS,
                                  preferred_element_type=jnp.float32)
        o_ref[...] = acc + b_ref[...]


def _forward_noacc(x, w, b, *, tm, kc):
    B, K = x.shape
    _, N = w.shape
    tn = N // 2
    nc = K // kc
    nt = B // tm
    grid_i = nc + nt - 1

    kern = functools.partial(_phased_kernel_noacc, nc=nc, kc=kc, nt=nt)

    def xa_map(j, i):
        return (0, jnp.minimum(i, nc - 1))

    def x_map(j, i):
        return (jnp.clip(i - nc + 1, 1, nt - 1), 0)

    def w_map(j, i):
        return (jnp.minimum(i, nc - 1), j)

    def o_map(j, i):
        return (jnp.where(i < nc, 0, jnp.clip(i - nc + 1, 1, nt - 1)), j)

    return pl.pallas_call(
        kern,
        out_shape=jax.ShapeDtypeStruct((B, N), x.dtype),
        grid=(2, grid_i),
        in_specs=[
            pl.BlockSpec((tm, kc), xa_map),
            pl.BlockSpec((tm, K), x_map),
            pl.BlockSpec((kc, tn), w_map),
            pl.BlockSpec((1, tn), lambda j, i: (0, j)),
        ],
        out_specs=pl.BlockSpec((tm, tn), o_map),
        scratch_shapes=[
            pltpu.VMEM((K, tn), jnp.bfloat16),
        ],
        compiler_params=pltpu.CompilerParams(
            dimension_semantics=("parallel", "arbitrary"),
            vmem_limit_bytes=64 << 20,
        ),
    )(x, x, w, b)


def _forward(x, w, b, *, tm, kc, pa):
    B, K = x.shape
    _, N = w.shape
    tn = N // 2
    nc = K // kc
    nt = B // tm
    grid_i = nc + nt

    kern = functools.partial(_phased_kernel, nc=nc, kc=kc, pa=pa, nt=nt)

    def xa_map(j, i):
        return (0, jnp.minimum(i, nc - 1))

    def x_map(j, i):
        return (jnp.clip(i - (nc - pa), pa, nt - 1), 0)

    def w_map(j, i):
        return (jnp.minimum(i, nc - 1), j)

    def o_map(j, i):
        return (jnp.where(i >= nc + nt - pa,
                          i - (nc + nt - pa),
                          jnp.clip(i - (nc - pa), pa, nt - 1)), j)

    return pl.pallas_call(
        kern,
        out_shape=jax.ShapeDtypeStruct((B, N), x.dtype),
        grid=(2, grid_i),
        in_specs=[
            pl.BlockSpec((pa * tm, kc), xa_map),
            pl.BlockSpec((tm, K), x_map),
            pl.BlockSpec((kc, tn), w_map),
            pl.BlockSpec((1, tn), lambda j, i: (0, j)),
        ],
        out_specs=pl.BlockSpec((tm, tn), o_map),
        scratch_shapes=[
            pltpu.VMEM((K, tn), jnp.bfloat16),
            pltpu.VMEM((pa * tm, tn), jnp.float32),
        ],
        compiler_params=pltpu.CompilerParams(
            dimension_semantics=("parallel", "arbitrary"),
            vmem_limit_bytes=64 << 20,
        ),
    )(x, x, w, b)


def kernel(x, w, b):
    B, K = x.shape
    K2, N = w.shape
    assert K == K2, (K, K2)
    assert B % 256 == 0 and K % 512 == 0 and N % 512 == 0, (B, K, N)
    return _forward_noacc(x, w, b, tm=512, kc=512)


# speedup vs baseline: 1.0962x; 1.0226x over previous
import functools

import jax
import jax.numpy as jnp
from jax.experimental import pallas as pl
from jax.experimental.pallas import tpu as pltpu

_DOT_DIMS = (((1,), (0,)), ((), ()))


def _phased_kernel(xa_ref, x_ref, w_ref, b_ref, o_ref, wb_ref, acc_ref,
                   *, nc, kc, pa, nt):
    i = pl.program_id(1)

    @pl.when(i < nc)
    def _phase_a():
        wc = w_ref[...].astype(jnp.bfloat16)
        wb_ref[pl.ds(i * kc, kc), :] = wc
        part = jax.lax.dot_general(xa_ref[...].astype(jnp.bfloat16), wc,
                                   dimension_numbers=_DOT_DIMS,
                                   preferred_element_type=jnp.float32)

        @pl.when(i == 0)
        def _():
            acc_ref[...] = part

        @pl.when(i > 0)
        def _():
            acc_ref[...] += part

    @pl.when(jnp.logical_and(i >= nc, i < nc + nt - pa))
    def _phase_b():
        acc = jax.lax.dot_general(x_ref[...].astype(jnp.bfloat16),
                                  wb_ref[...],
                                  dimension_numbers=_DOT_DIMS,
                                  preferred_element_type=jnp.float32)
        o_ref[...] = acc + b_ref[...]

    @pl.when(i >= nc + nt - pa)
    def _tail():
        r = (i - (nc + nt - pa)) * o_ref.shape[0]
        o_ref[...] = acc_ref[pl.ds(r, o_ref.shape[0]), :] + b_ref[...]


def _phased_kernel_noacc(xa_ref, x_ref, w_ref, b_ref, o_ref, wb_ref,
                         *, nc, kc, nt):
    i = pl.program_id(1)

    @pl.when(i < nc)
    def _phase_a():
        wc = w_ref[...].astype(jnp.bfloat16)
        wb_ref[pl.ds(i * kc, kc), :] = wc
        part = jax.lax.dot_general(xa_ref[...].astype(jnp.bfloat16), wc,
                                   dimension_numbers=_DOT_DIMS,
                                   preferred_element_type=jnp.float32)

        @pl.when(i == 0)
        def _():
            o_ref[...] = part + b_ref[...]

        @pl.when(i > 0)
        def _():
            o_ref[...] += part

    @pl.when(i >= nc)
    def _phase_b():
        acc = jax.lax.dot_general(x_ref[...].astype(jnp.bfloat16),
                                  wb_ref[...],
                                  dimension_numbers=_DOT_DIMS,
                                  preferred_element_type=jnp.float32)
        o_ref[...] = acc + b_ref[...]


def _forward_noacc(x, w, b, *, tm, kc):
    B, K = x.shape
    _, N = w.shape
    tn = N // 2
    nc = K // kc
    nt = B // tm
    grid_i = nc + nt - 1

    kern = functools.partial(_phased_kernel_noacc, nc=nc, kc=kc, nt=nt)

    def xa_map(j, i):
        return (0, jnp.minimum(i, nc - 1))

    def x_map(j, i):
        return (jnp.clip(i - nc + 1, 1, nt - 1), 0)

    def w_map(j, i):
        return (jnp.minimum(i, nc - 1), j)

    def o_map(j, i):
        return (jnp.where(i < nc, 0, jnp.clip(i - nc + 1, 1, nt - 1)), j)

    return pl.pallas_call(
        kern,
        out_shape=jax.ShapeDtypeStruct((B, N), x.dtype),
        grid=(2, grid_i),
        in_specs=[
            pl.BlockSpec((tm, kc), xa_map),
            pl.BlockSpec((tm, K), x_map),
            pl.BlockSpec((kc, tn), w_map),
            pl.BlockSpec((1, tn), lambda j, i: (0, j)),
        ],
        out_specs=pl.BlockSpec((tm, tn), o_map),
        scratch_shapes=[
            pltpu.VMEM((K, tn), jnp.bfloat16),
        ],
        compiler_params=pltpu.CompilerParams(
            dimension_semantics=("parallel", "arbitrary"),
            vmem_limit_bytes=64 << 20,
        ),
    )(x, x, w, b)


def _forward(x, w, b, *, tm, kc, pa):
    B, K = x.shape
    _, N = w.shape
    tn = N // 2
    nc = K // kc
    nt = B // tm
    grid_i = nc + nt

    kern = functools.partial(_phased_kernel, nc=nc, kc=kc, pa=pa, nt=nt)

    def xa_map(j, i):
        return (0, jnp.minimum(i, nc - 1))

    def x_map(j, i):
        return (jnp.clip(i - (nc - pa), pa, nt - 1), 0)

    def w_map(j, i):
        return (jnp.minimum(i, nc - 1), j)

    def o_map(j, i):
        return (jnp.where(i >= nc + nt - pa,
                          i - (nc + nt - pa),
                          jnp.clip(i - (nc - pa), pa, nt - 1)), j)

    return pl.pallas_call(
        kern,
        out_shape=jax.ShapeDtypeStruct((B, N), x.dtype),
        grid=(2, grid_i),
        in_specs=[
            pl.BlockSpec((pa * tm, kc), xa_map),
            pl.BlockSpec((tm, K), x_map),
            pl.BlockSpec((kc, tn), w_map),
            pl.BlockSpec((1, tn), lambda j, i: (0, j)),
        ],
        out_specs=pl.BlockSpec((tm, tn), o_map),
        scratch_shapes=[
            pltpu.VMEM((K, tn), jnp.bfloat16),
            pltpu.VMEM((pa * tm, tn), jnp.float32),
        ],
        compiler_params=pltpu.CompilerParams(
            dimension_semantics=("parallel", "arbitrary"),
            vmem_limit_bytes=64 << 20,
        ),
    )(x, x, w, b)


def kernel(x, w, b):
    B, K = x.shape
    K2, N = w.shape
    assert K == K2, (K, K2)
    assert B % 256 == 0 and K % 512 == 0 and N % 512 == 0, (B, K, N)
    return _forward_noacc(x, w, b, tm=512, kc=1024)


# speedup vs baseline: 1.1034x; 1.0065x over previous
import functools

import jax
import jax.numpy as jnp
from jax.experimental import pallas as pl
from jax.experimental.pallas import tpu as pltpu

_DOT_DIMS = (((1,), (0,)), ((), ()))


def _phased_kernel(xa_ref, x_ref, w_ref, b_ref, o_ref, wb_ref, acc_ref,
                   *, nc, kc, pa, nt):
    i = pl.program_id(1)

    @pl.when(i < nc)
    def _phase_a():
        wc = w_ref[...].astype(jnp.bfloat16)
        wb_ref[pl.ds(i * kc, kc), :] = wc
        part = jax.lax.dot_general(xa_ref[...].astype(jnp.bfloat16), wc,
                                   dimension_numbers=_DOT_DIMS,
                                   preferred_element_type=jnp.float32)

        @pl.when(i == 0)
        def _():
            acc_ref[...] = part

        @pl.when(i > 0)
        def _():
            acc_ref[...] += part

    @pl.when(jnp.logical_and(i >= nc, i < nc + nt - pa))
    def _phase_b():
        acc = jax.lax.dot_general(x_ref[...].astype(jnp.bfloat16),
                                  wb_ref[...],
                                  dimension_numbers=_DOT_DIMS,
                                  preferred_element_type=jnp.float32)
        o_ref[...] = acc + b_ref[...]

    @pl.when(i >= nc + nt - pa)
    def _tail():
        r = (i - (nc + nt - pa)) * o_ref.shape[0]
        o_ref[...] = acc_ref[pl.ds(r, o_ref.shape[0]), :] + b_ref[...]


def _phased_kernel_noacc(x_ref, w_ref, b_ref, o_ref, wb_ref,
                         *, nc, kc, nt):
    i = pl.program_id(1)

    @pl.when(i < nc)
    def _phase_a():
        wc = w_ref[...].astype(jnp.bfloat16)
        wb_ref[pl.ds(i * kc, kc), :] = wc
        xs = x_ref[:, pl.ds(i * kc, kc)].astype(jnp.bfloat16)
        part = jax.lax.dot_general(xs, wc,
                                   dimension_numbers=_DOT_DIMS,
                                   preferred_element_type=jnp.float32)

        @pl.when(i == 0)
        def _():
            o_ref[...] = part + b_ref[...]

        @pl.when(i > 0)
        def _():
            o_ref[...] += part

    @pl.when(i >= nc)
    def _phase_b():
        acc = jax.lax.dot_general(x_ref[...].astype(jnp.bfloat16),
                                  wb_ref[...],
                                  dimension_numbers=_DOT_DIMS,
                                  preferred_element_type=jnp.float32)
        o_ref[...] = acc + b_ref[...]


def _forward_noacc(x, w, b, *, tm, kc):
    B, K = x.shape
    _, N = w.shape
    tn = N // 2
    nc = K // kc
    nt = B // tm
    grid_i = nc + nt - 1

    kern = functools.partial(_phased_kernel_noacc, nc=nc, kc=kc, nt=nt)

    def x_map(j, i):
        return (jnp.clip(i - nc + 1, 0, nt - 1), 0)

    def w_map(j, i):
        return (jnp.minimum(i, nc - 1), j)

    def o_map(j, i):
        return (jnp.where(i < nc, 0, jnp.clip(i - nc + 1, 1, nt - 1)), j)

    return pl.pallas_call(
        kern,
        out_shape=jax.ShapeDtypeStruct((B, N), x.dtype),
        grid=(2, grid_i),
        in_specs=[
            pl.BlockSpec((tm, K), x_map),
            pl.BlockSpec((kc, tn), w_map),
            pl.BlockSpec((1, tn), lambda j, i: (0, j)),
        ],
        out_specs=pl.BlockSpec((tm, tn), o_map),
        scratch_shapes=[
            pltpu.VMEM((K, tn), jnp.bfloat16),
        ],
        compiler_params=pltpu.CompilerParams(
            dimension_semantics=("parallel", "arbitrary"),
            vmem_limit_bytes=64 << 20,
        ),
    )(x, w, b)


def _forward(x, w, b, *, tm, kc, pa):
    B, K = x.shape
    _, N = w.shape
    tn = N // 2
    nc = K // kc
    nt = B // tm
    grid_i = nc + nt

    kern = functools.partial(_phased_kernel, nc=nc, kc=kc, pa=pa, nt=nt)

    def xa_map(j, i):
        return (0, jnp.minimum(i, nc - 1))

    def x_map(j, i):
        return (jnp.clip(i - (nc - pa), pa, nt - 1), 0)

    def w_map(j, i):
        return (jnp.minimum(i, nc - 1), j)

    def o_map(j, i):
        return (jnp.where(i >= nc + nt - pa,
                          i - (nc + nt - pa),
                          jnp.clip(i - (nc - pa), pa, nt - 1)), j)

    return pl.pallas_call(
        kern,
        out_shape=jax.ShapeDtypeStruct((B, N), x.dtype),
        grid=(2, grid_i),
        in_specs=[
            pl.BlockSpec((pa * tm, kc), xa_map),
            pl.BlockSpec((tm, K), x_map),
            pl.BlockSpec((kc, tn), w_map),
            pl.BlockSpec((1, tn), lambda j, i: (0, j)),
        ],
        out_specs=pl.BlockSpec((tm, tn), o_map),
        scratch_shapes=[
            pltpu.VMEM((K, tn), jnp.bfloat16),
            pltpu.VMEM((pa * tm, tn), jnp.float32),
        ],
        compiler_params=pltpu.CompilerParams(
            dimension_semantics=("parallel", "arbitrary"),
            vmem_limit_bytes=64 << 20,
        ),
    )(x, x, w, b)


def kernel(x, w, b):
    B, K = x.shape
    K2, N = w.shape
    assert K == K2, (K, K2)
    assert B % 256 == 0 and K % 512 == 0 and N % 512 == 0, (B, K, N)
    return _forward_noacc(x, w, b, tm=512, kc=1024)


# speedup vs baseline: 1.1047x; 1.0012x over previous
import functools

import jax
import jax.numpy as jnp
from jax.experimental import pallas as pl
from jax.experimental.pallas import tpu as pltpu

_DOT_DIMS = (((1,), (0,)), ((), ()))


def _phased_kernel(x_ref, w_ref, b_ref, o_ref, wb_ref, *, nc, kc, nt):
    i = pl.program_id(1)

    @pl.when(i < nc)
    def _phase_a():
        wc = w_ref[...].astype(jnp.bfloat16)
        wb_ref[pl.ds(i * kc, kc), :] = wc
        xs = x_ref[:, pl.ds(i * kc, kc)].astype(jnp.bfloat16)
        part = jax.lax.dot_general(xs, wc,
                                   dimension_numbers=_DOT_DIMS,
                                   preferred_element_type=jnp.float32)

        @pl.when(i == 0)
        def _():
            o_ref[...] = part + b_ref[...]

        @pl.when(i > 0)
        def _():
            o_ref[...] += part

    @pl.when(i >= nc)
    def _phase_b():
        acc = jax.lax.dot_general(x_ref[...].astype(jnp.bfloat16),
                                  wb_ref[...],
                                  dimension_numbers=_DOT_DIMS,
                                  preferred_element_type=jnp.float32)
        o_ref[...] = acc + b_ref[...]


def _forward(x, w, b, *, tm, kc):
    B, K = x.shape
    _, N = w.shape
    tn = N // 2
    nc = K // kc
    nt = B // tm
    grid_i = nc + nt - 1

    kern = functools.partial(_phased_kernel, nc=nc, kc=kc, nt=nt)

    def x_map(j, i):
        return (jnp.clip(i - nc + 1, 0, nt - 1), 0)

    def w_map(j, i):
        return (jnp.minimum(i, nc - 1), j)

    def o_map(j, i):
        return (jnp.where(i < nc, 0, jnp.clip(i - nc + 1, 0, nt - 1)), j)

    return pl.pallas_call(
        kern,
        out_shape=jax.ShapeDtypeStruct((B, N), x.dtype),
        grid=(2, grid_i),
        in_specs=[
            pl.BlockSpec((tm, K), x_map),
            pl.BlockSpec((kc, tn), w_map),
            pl.BlockSpec((1, tn), lambda j, i: (0, j)),
        ],
        out_specs=pl.BlockSpec((tm, tn), o_map),
        scratch_shapes=[
            pltpu.VMEM((K, tn), jnp.bfloat16),
        ],
        compiler_params=pltpu.CompilerParams(
            dimension_semantics=("parallel", "arbitrary"),
            vmem_limit_bytes=64 << 20,
        ),
    )(x, w, b)


def kernel(x, w, b):
    B, K = x.shape
    K2, N = w.shape
    assert K == K2, (K, K2)
    assert B % 512 == 0 and K % 1024 == 0 and N % 512 == 0, (B, K, N)
    return _forward(x, w, b, tm=512, kc=1024)
